# baseline (speedup 1.0000x reference)
import functools

import jax
import jax.numpy as jnp
from jax import lax
from jax.experimental import pallas as pl
from jax.experimental.pallas import tpu as pltpu
from jax.experimental.pallas import tpu_sc as plsc

N_NODES = 10000
N_EDGES = 320000
H = 64

NC = 2
NS = 16
NW = NC * NS
CHUNK = 128
NCHUNK = 80
SCH = 128
NSCH = 80
EPT = CHUNK * NCHUNK
E_PAD = EPT * NW
DUMP_ROW = N_NODES
NACC = 10112
ROWS_PER_TILE = NACC // NS
D = 128
BLK = 1000
NBLK = N_NODES // BLK
EBLK = 4096
NEBLK = E_PAD // EBLK

_MESH = plsc.VectorSubcoreMesh(core_axis_name="c", subcore_axis_name="s")


@functools.partial(
    pl.kernel,
    mesh=_MESH,
    out_type=jax.ShapeDtypeStruct((NC, NACC, D), jnp.float32),
    scratch_types=[
        pltpu.VMEM((NSCH, SCH), jnp.int32),
        pltpu.VMEM((NSCH, SCH), jnp.int32),
        pltpu.VMEM((SCH, D), jnp.float32),
        pltpu.VMEM_SHARED((NACC, D), jnp.float32),
    ],
)
def _SCATTER(src2d, rec2d, table, out, idxs_v, idxr_v, rows0, acc_sh):
    cid = lax.axis_index("c")
    sid = lax.axis_index("s")
    wid = cid * NS + sid

    pltpu.sync_copy(src2d.at[wid], idxs_v)
    pltpu.sync_copy(rec2d.at[wid], idxr_v)

    def zrow(r, carry):
        for cc in range(0, D, 16):
            rows0[r, pl.ds(cc, 16)] = jnp.zeros((16,), jnp.float32)
        return carry

    lax.fori_loop(0, SCH, zrow, 0)
    abase = sid * ROWS_PER_TILE
    for kk in range(ROWS_PER_TILE // SCH):
        pltpu.sync_copy(rows0, acc_sh.at[pl.ds(abase + kk * SCH, SCH)])
    _tail = ROWS_PER_TILE % SCH
    if _tail:
        pltpu.sync_copy(
            rows0.at[pl.ds(0, _tail)],
            acc_sh.at[pl.ds(abase + (ROWS_PER_TILE // SCH) * SCH, _tail)])
    plsc.subcore_barrier()

    def body(j, carry):
        pltpu.sync_copy(table.at[idxs_v.at[j]], rows0)
        pltpu.sync_copy(rows0, acc_sh.at[idxr_v.at[j]], add=True)
        return carry

    lax.fori_loop(0, NSCH, body, 0)

    plsc.subcore_barrier()
    pltpu.sync_copy(
        acc_sh.at[pl.ds(sid * ROWS_PER_TILE, ROWS_PER_TILE)],
        out.at[cid, pl.ds(sid * ROWS_PER_TILE, ROWS_PER_TILE)])


@functools.partial(
    pl.kernel,
    mesh=_MESH,
    out_type=jax.ShapeDtypeStruct((E_PAD, D), jnp.float32),
    scratch_types=[
        pltpu.VMEM((NCHUNK, CHUNK), jnp.int32),
        pltpu.VMEM((NCHUNK, CHUNK), jnp.int32),
        pltpu.VMEM((CHUNK, D), jnp.float32),
        pltpu.VMEM((CHUNK, D), jnp.float32),
        pltpu.VMEM((CHUNK, D), jnp.float32),
        pltpu.VMEM((CHUNK, D), jnp.float32),
        pltpu.VMEM((CHUNK, D), jnp.float32),
        pltpu.VMEM((CHUNK, D), jnp.float32),
        pltpu.SemaphoreType.DMA,
        pltpu.SemaphoreType.DMA,
    ],
)
def _EDGE_UPDATE(src2d, rec2d, ts, tr, q, e_out,
                 idxs_v, idxr_v, a0, b0, q0, a1, b1, q1, sem0, sem1):
    cid = lax.axis_index("c")
    sid = lax.axis_index("s")
    wid = cid * NS + sid

    row0 = wid * NCHUNK
    pltpu.sync_copy(src2d.at[pl.ds(row0, NCHUNK)], idxs_v)
    pltpu.sync_copy(rec2d.at[pl.ds(row0, NCHUNK)], idxr_v)

    ebase = wid * EPT

    def issue(j, ra, rb, rq, sem):
        pltpu.async_copy(ts.at[idxs_v.at[j]], ra, sem)
        pltpu.async_copy(tr.at[idxr_v.at[j]], rb, sem)
        pltpu.async_copy(q.at[pl.ds(ebase + j * CHUNK, CHUNK)], rq, sem)

    def wait_all(j, ra, rb, rq, sem):
        pltpu.make_async_copy(ts.at[idxs_v.at[j]], ra, sem).wait()
        pltpu.make_async_copy(tr.at[idxr_v.at[j]], rb, sem).wait()
        pltpu.make_async_copy(q.at[pl.ds(ebase + j * CHUNK, CHUNK)],
                              rq, sem).wait()

    def compute_store(j, ra, rb, rq):
        def erow(r2, carry2):
            for dr in range(2):
                r = 2 * r2 + dr
                for cc in range(H, D, 16):
                    s = (ra[r, pl.ds(cc, 16)] + rb[r, pl.ds(cc, 16)]
                         + rq[r, pl.ds(cc, 16)])
                    ra[r, pl.ds(cc, 16)] = s.astype(jnp.bfloat16).astype(
                        jnp.float32)
            return carry2

        lax.fori_loop(0, CHUNK // 2, erow, 0)
        pltpu.sync_copy(ra, e_out.at[pl.ds(ebase + j * CHUNK, CHUNK)])

    issue(0, a0, b0, q0, sem0)

    def pair(i, carry):
        j0 = 2 * i
        j1 = j0 + 1
        issue(j1, a1, b1, q1, sem1)
        wait_all(j0, a0, b0, q0, sem0)
        compute_store(j0, a0, b0, q0)

        @pl.when(i < NCHUNK // 2 - 1)
        def _():
            issue(j0 + 2, a0, b0, q0, sem0)

        wait_all(j1, a1, b1, q1, sem1)
        compute_store(j1, a1, b1, q1)
        return carry

    lax.fori_loop(0, NCHUNK // 2, pair, 0)


def _dd(a, b):
    return jnp.dot(a, b, preferred_element_type=jnp.float32)


def _rbf(x):
    return x.astype(jnp.bfloat16).astype(jnp.float32)


def _hp(a, b):
    return jnp.dot(a, _rbf(b), preferred_element_type=jnp.float32,
                   precision=jax.lax.Precision.HIGHEST)


def _acc_lo(ref):
    row0 = pl.multiple_of(pl.program_id(0) * BLK, 8)
    return ref[pl.ds(row0, BLK), 0:H]


def _acc_hi(ref):
    row0 = pl.multiple_of(pl.program_id(0) * BLK, 8)
    return ref[pl.ds(row0, BLK), H:D]


def _tce_body(e0, We, be, T0):
    ev = _rbf(_dd(e0[...], We[...]) + be[...])
    lo = jnp.concatenate(
        [jnp.ones((EBLK, 1), jnp.float32),
         jnp.zeros((EBLK, H - 1), jnp.float32)], axis=1)
    T0[:, 0:H] = lo
    T0[:, H:D] = ev


def _tcq_body(ecur, We3, beu, Q):
    qv = _dd(ecur[:, H:D], We3[...]) + beu[...]
    Q[:, 0:H] = jnp.zeros((EBLK, H), jnp.float32)
    Q[:, H:D] = qv


def _tc0_body(h0, p0, Wh, bh, Wp, bp, Wm0, Sp0, Sp1, h1, p1, A0, SD):
    hv = _dd(h0[...], Wh[...]) + bh[...]
    pv = _dd(p0[...], Wp[...]) + bp[...]
    h1[...] = hv
    p1[...] = pv
    Wm = Wm0[...]
    A0[:, 0:H] = _dd(hv, Wm[0:64]) + _dd(pv, Wm[64:128])
    A0[:, H:D] = jnp.zeros((BLK, H), jnp.float32)
    row0 = pl.multiple_of(pl.program_id(0) * BLK, 8)
    SD[...] = Sp0[pl.ds(row0, BLK), :] + Sp1[pl.ds(row0, BLK), :]


def _tcb_body(h, p, St, SD, acc0, acc1,
              Wm, bm, Wu, bu, Weu, Wpm,
              h_new, TS, TR, P2):
    hv, pv = h[...], p[...]
    Wmv, Wuv, Weuv, Wpmv = Wm[...], Wu[...], Weu[...], Wpm[...]
    indeg = SD[:, 0:1]

    B = _dd(hv, Wmv[128:192]) + _dd(pv, Wmv[192:256])
    h_agg = (_acc_lo(acc0) + _acc_lo(acc1) + _hp(St[...], Wmv[256:320])
             + indeg * (B + bm[...]))
    hn = _dd(hv, Wuv[0:64]) + _dd(h_agg, Wuv[64:128]) + bu[...]
    h_new[...] = hn

    U2 = _dd(hn, Weuv[0:64])
    R2 = _dd(hn, Weuv[64:128])
    TS[:, 0:H] = _dd(pv, Wpmv[0:64])
    TS[:, H:D] = U2
    TR[:, 0:H] = jnp.zeros((BLK, H), jnp.float32)
    TR[:, H:D] = R2
    P2[...] = _dd(pv, Wpmv[64:128])


def _tcc_body(p, h_new, SD, acc0, acc1, P2, Wpm, bpm, Wpu, bpu, Wm_next,
              p_new, A_next, St_next):
    pv = p[...]
    indeg = SD[:, 0:1]
    stn = _acc_hi(acc0) + _acc_hi(acc1)
    St_next[...] = stn
    p_agg = (_acc_lo(acc0) + _acc_lo(acc1) + _hp(stn, Wpm[...][128:192])
             + indeg * (P2[...] + bpm[...]))
    Wpuv = Wpu[...]
    pn = _dd(pv, Wpuv[0:64]) + _dd(p_agg, Wpuv[64:128]) + bpu[...]
    p_new[...] = pn
    Wmn = Wm_next[...]
    A_next[:, 0:H] = _dd(h_new[...], Wmn[0:64]) + _dd(pn, Wmn[64:128])
    A_next[:, H:D] = jnp.zeros((BLK, H), jnp.float32)


_NH = jax.ShapeDtypeStruct((N_NODES, H), jnp.float32)
_TABLE = jax.ShapeDtypeStruct((N_NODES, D), jnp.float32)
_ESTREAM = jax.ShapeDtypeStruct((E_PAD, D), jnp.float32)


def _blk(w):
    return pl.BlockSpec((BLK, w), lambda i: (i, 0))


def _eblk(w):
    return pl.BlockSpec((EBLK, w), lambda i: (i, 0))


def _full(*shape):
    return pl.BlockSpec(shape, lambda i: (0,) * len(shape))


_TCE = pl.pallas_call(
    _tce_body,
    grid=(NEBLK,),
    in_specs=[_eblk(16), _full(16, H), _full(1, H)],
    out_specs=_eblk(D),
    out_shape=_ESTREAM,
)

_TCQ = pl.pallas_call(
    _tcq_body,
    grid=(NEBLK,),
    in_specs=[_eblk(D), _full(H, H), _full(1, H)],
    out_specs=_eblk(D),
    out_shape=_ESTREAM,
)

_TC0 = pl.pallas_call(
    _tc0_body,
    grid=(NBLK,),
    in_specs=[_blk(128), _blk(16), _full(128, H), _full(1, H),
              _full(16, H), _full(1, H), _full(320, H),
              _full(NACC, D), _full(NACC, D)],
    out_specs=[_blk(H), _blk(H), _blk(D), _blk(D)],
    out_shape=[_NH, _NH, _TABLE, jax.ShapeDtypeStruct((N_NODES, D),
                                                      jnp.float32)],
)

_TCB = pl.pallas_call(
    _tcb_body,
    grid=(NBLK,),
    in_specs=[_blk(H), _blk(H), _blk(H), _blk(D),
              _full(NACC, D), _full(NACC, D),
              _full(320, H), _full(1, H), _full(128, H), _full(1, H),
              _full(192, H), _full(192, H)],
    out_specs=[_blk(H), _blk(D), _blk(D), _blk(H)],
    out_shape=[_NH, _TABLE, _TABLE, _NH],
)

_TCC = pl.pallas_call(
    _tcc_body,
    grid=(NBLK,),
    in_specs=[_blk(H), _blk(H), _blk(D),
              _full(NACC, D), _full(NACC, D), _blk(H),
              _full(192, H), _full(1, H), _full(128, H), _full(1, H),
              _full(320, H)],
    out_specs=[_blk(H), _blk(D), _blk(H)],
    out_shape=[_NH, _TABLE, _NH],
)


def kernel(h, e, p, edge_index, params):
    send = edge_index[0].astype(jnp.int32)
    rec = edge_index[1].astype(jnp.int32)
    n_pad = E_PAD - N_EDGES
    send_pad = jnp.concatenate([send, jnp.zeros((n_pad,), jnp.int32)])
    rec_pad = jnp.concatenate([rec, jnp.full((n_pad,), DUMP_ROW, jnp.int32)])
    send2d = send_pad.reshape(E_PAD // CHUNK, CHUNK)
    rec2d = rec_pad.reshape(E_PAD // CHUNK, CHUNK)
    sendS = send_pad.reshape(NW, NSCH, SCH)
    recS = rec_pad.reshape(NW, NSCH, SCH)
    iotaS = jnp.arange(E_PAD, dtype=jnp.int32).reshape(NW, NSCH, SCH)

    b2 = lambda x: x.reshape(1, H)

    e_pad = jnp.concatenate([e, jnp.zeros((n_pad, 16), jnp.float32)])
    estream = _TCE(e_pad, params["We"], b2(params["be"]))
    Sp = _SCATTER(iotaS, recS, estream)

    h1, p1, A, SD = _TC0(h, p, params["Wh"], b2(params["bh"]),
                         params["Wp"], b2(params["bp"]),
                         params["layers"][0]["Wm"], Sp[0], Sp[1])
    St = SD[:, H:D]
    hcur, pcur = h1, p1

    n_layers = len(params["layers"])
    for li, lp in enumerate(params["layers"]):
        acc_h = _SCATTER(sendS, recS, A)
        hcur, TS, TR, P2 = _TCB(
            hcur, pcur, St, SD, acc_h[0], acc_h[1],
            lp["Wm"], b2(lp["bm"]), lp["Wu"], b2(lp["bu"]),
            lp["Weu"], lp["Wpm"])
        Q = _TCQ(estream, lp["Weu"][128:192], b2(lp["beu"]))
        estream = _EDGE_UPDATE(send2d, rec2d, TS, TR, Q)
        acc2 = _SCATTER(iotaS, recS, estream)
        Wm_next = params["layers"][(li + 1) % n_layers]["Wm"]
        pcur, A, St = _TCC(pcur, hcur, SD, acc2[0], acc2[1], P2,
                           lp["Wpm"], b2(lp["bpm"]), lp["Wpu"], b2(lp["bpu"]),
                           Wm_next)

    return (hcur, pcur)


# speedup vs baseline: 1.0046x; 1.0046x over previous
import functools

import jax
import jax.numpy as jnp
from jax import lax
from jax.experimental import pallas as pl
from jax.experimental.pallas import tpu as pltpu
from jax.experimental.pallas import tpu_sc as plsc

N_NODES = 10000
N_EDGES = 320000
H = 64

NC = 2
NS = 16
NW = NC * NS
CHUNK = 128
NCHUNK = 80
SCH = 128
NSCH = 80
EPT = CHUNK * NCHUNK
E_PAD = EPT * NW
DUMP_ROW = N_NODES
NACC = 10112
ROWS_PER_TILE = NACC // NS
D = 128
BLK = 1000
NBLK = N_NODES // BLK
EBLK = 4096
NEBLK = E_PAD // EBLK

_MESH = plsc.VectorSubcoreMesh(core_axis_name="c", subcore_axis_name="s")


@functools.partial(
    pl.kernel,
    mesh=_MESH,
    out_type=jax.ShapeDtypeStruct((NC, NACC, D), jnp.float32),
    scratch_types=[
        pltpu.VMEM((NSCH, SCH), jnp.int32),
        pltpu.VMEM((NSCH, SCH), jnp.int32),
        pltpu.VMEM((SCH, D), jnp.float32),
        pltpu.VMEM_SHARED((NACC, D), jnp.float32),
    ],
)
def _SCATTER(src2d, rec2d, table, out, idxs_v, idxr_v, rows0, acc_sh):
    cid = lax.axis_index("c")
    sid = lax.axis_index("s")
    wid = cid * NS + sid

    pltpu.sync_copy(src2d.at[wid], idxs_v)
    pltpu.sync_copy(rec2d.at[wid], idxr_v)

    def zrow(r, carry):
        for cc in range(0, D, 16):
            rows0[r, pl.ds(cc, 16)] = jnp.zeros((16,), jnp.float32)
        return carry

    lax.fori_loop(0, SCH, zrow, 0)
    abase = sid * ROWS_PER_TILE
    for kk in range(ROWS_PER_TILE // SCH):
        pltpu.sync_copy(rows0, acc_sh.at[pl.ds(abase + kk * SCH, SCH)])
    _tail = ROWS_PER_TILE % SCH
    if _tail:
        pltpu.sync_copy(
            rows0.at[pl.ds(0, _tail)],
            acc_sh.at[pl.ds(abase + (ROWS_PER_TILE // SCH) * SCH, _tail)])
    plsc.subcore_barrier()

    def body(j, carry):
        pltpu.sync_copy(table.at[idxs_v.at[j]], rows0)
        pltpu.sync_copy(rows0, acc_sh.at[idxr_v.at[j]], add=True)
        return carry

    lax.fori_loop(0, NSCH, body, 0)

    plsc.subcore_barrier()
    pltpu.sync_copy(
        acc_sh.at[pl.ds(sid * ROWS_PER_TILE, ROWS_PER_TILE)],
        out.at[cid, pl.ds(sid * ROWS_PER_TILE, ROWS_PER_TILE)])


@functools.partial(
    pl.kernel,
    mesh=_MESH,
    out_type=jax.ShapeDtypeStruct((E_PAD, D), jnp.float32),
    scratch_types=[
        pltpu.VMEM((NCHUNK, CHUNK), jnp.int32),
        pltpu.VMEM((NCHUNK, CHUNK), jnp.int32),
        pltpu.VMEM((CHUNK, D), jnp.float32),
        pltpu.VMEM((CHUNK, D), jnp.float32),
        pltpu.VMEM((CHUNK, H), jnp.float32),
        pltpu.VMEM((CHUNK, D), jnp.float32),
        pltpu.VMEM((CHUNK, D), jnp.float32),
        pltpu.VMEM((CHUNK, H), jnp.float32),
        pltpu.SemaphoreType.DMA,
        pltpu.SemaphoreType.DMA,
    ],
)
def _EDGE_UPDATE(src2d, rec2d, ts, tr, q, e_out,
                 idxs_v, idxr_v, a0, b0, q0, a1, b1, q1, sem0, sem1):
    cid = lax.axis_index("c")
    sid = lax.axis_index("s")
    wid = cid * NS + sid

    row0 = wid * NCHUNK
    pltpu.sync_copy(src2d.at[pl.ds(row0, NCHUNK)], idxs_v)
    pltpu.sync_copy(rec2d.at[pl.ds(row0, NCHUNK)], idxr_v)

    ebase = wid * EPT

    def issue(j, ra, rb, rq, sem):
        pltpu.async_copy(ts.at[idxs_v.at[j]], ra, sem)
        pltpu.async_copy(tr.at[idxr_v.at[j]], rb, sem)
        pltpu.async_copy(q.at[pl.ds(ebase + j * CHUNK, CHUNK)], rq, sem)

    def wait_all(j, ra, rb, rq, sem):
        pltpu.make_async_copy(ts.at[idxs_v.at[j]], ra, sem).wait()
        pltpu.make_async_copy(tr.at[idxr_v.at[j]], rb, sem).wait()
        pltpu.make_async_copy(q.at[pl.ds(ebase + j * CHUNK, CHUNK)],
                              rq, sem).wait()

    def compute_store(j, ra, rb, rq):
        def erow(r2, carry2):
            for dr in range(2):
                r = 2 * r2 + dr
                for cc in range(H, D, 16):
                    s = (ra[r, pl.ds(cc, 16)] + rb[r, pl.ds(cc, 16)]
                         + rq[r, pl.ds(cc - H, 16)])
                    ra[r, pl.ds(cc, 16)] = s.astype(jnp.bfloat16).astype(
                        jnp.float32)
            return carry2

        lax.fori_loop(0, CHUNK // 2, erow, 0)
        pltpu.sync_copy(ra, e_out.at[pl.ds(ebase + j * CHUNK, CHUNK)])

    issue(0, a0, b0, q0, sem0)

    def pair(i, carry):
        j0 = 2 * i
        j1 = j0 + 1
        issue(j1, a1, b1, q1, sem1)
        wait_all(j0, a0, b0, q0, sem0)
        compute_store(j0, a0, b0, q0)

        @pl.when(i < NCHUNK // 2 - 1)
        def _():
            issue(j0 + 2, a0, b0, q0, sem0)

        wait_all(j1, a1, b1, q1, sem1)
        compute_store(j1, a1, b1, q1)
        return carry

    lax.fori_loop(0, NCHUNK // 2, pair, 0)


def _dd(a, b):
    return jnp.dot(a, b, preferred_element_type=jnp.float32)


def _rbf(x):
    return x.astype(jnp.bfloat16).astype(jnp.float32)


def _hp(a, b):
    return jnp.dot(a, _rbf(b), preferred_element_type=jnp.float32,
                   precision=jax.lax.Precision.HIGHEST)


def _acc_lo(ref):
    row0 = pl.multiple_of(pl.program_id(0) * BLK, 8)
    return ref[pl.ds(row0, BLK), 0:H]


def _acc_hi(ref):
    row0 = pl.multiple_of(pl.program_id(0) * BLK, 8)
    return ref[pl.ds(row0, BLK), H:D]


def _tce_body(e0, We, be, T0):
    ev = _rbf(_dd(e0[...], We[...]) + be[...])
    lo = jnp.concatenate(
        [jnp.ones((EBLK, 1), jnp.float32),
         jnp.zeros((EBLK, H - 1), jnp.float32)], axis=1)
    T0[:, 0:H] = lo
    T0[:, H:D] = ev


def _tcq_body(ecur, We3, beu, Q):
    Q[...] = _dd(ecur[:, H:D], We3[...]) + beu[...]


def _tc0_body(h0, p0, Wh, bh, Wp, bp, Wm0, Sp0, Sp1, h1, p1, A0, SD):
    hv = _dd(h0[...], Wh[...]) + bh[...]
    pv = _dd(p0[...], Wp[...]) + bp[...]
    h1[...] = hv
    p1[...] = pv
    Wm = Wm0[...]
    A0[:, 0:H] = _dd(hv, Wm[0:64]) + _dd(pv, Wm[64:128])
    A0[:, H:D] = jnp.zeros((BLK, H), jnp.float32)
    row0 = pl.multiple_of(pl.program_id(0) * BLK, 8)
    SD[...] = Sp0[pl.ds(row0, BLK), :] + Sp1[pl.ds(row0, BLK), :]


def _tcb_body(h, p, St, SD, acc0, acc1,
              Wm, bm, Wu, bu, Weu, Wpm,
              h_new, TS, TR, P2):
    hv, pv = h[...], p[...]
    Wmv, Wuv, Weuv, Wpmv = Wm[...], Wu[...], Weu[...], Wpm[...]
    indeg = SD[:, 0:1]

    B = _dd(hv, Wmv[128:192]) + _dd(pv, Wmv[192:256])
    h_agg = (_acc_lo(acc0) + _acc_lo(acc1) + _hp(St[...], Wmv[256:320])
             + indeg * (B + bm[...]))
    hn = _dd(hv, Wuv[0:64]) + _dd(h_agg, Wuv[64:128]) + bu[...]
    h_new[...] = hn

    U2 = _dd(hn, Weuv[0:64])
    R2 = _dd(hn, Weuv[64:128])
    TS[:, 0:H] = _dd(pv, Wpmv[0:64])
    TS[:, H:D] = U2
    TR[:, 0:H] = jnp.zeros((BLK, H), jnp.float32)
    TR[:, H:D] = R2
    P2[...] = _dd(pv, Wpmv[64:128])


def _tcc_body(p, h_new, SD, acc0, acc1, P2, Wpm, bpm, Wpu, bpu, Wm_next,
              p_new, A_next, St_next):
    pv = p[...]
    indeg = SD[:, 0:1]
    stn = _acc_hi(acc0) + _acc_hi(acc1)
    St_next[...] = stn
    p_agg = (_acc_lo(acc0) + _acc_lo(acc1) + _hp(stn, Wpm[...][128:192])
             + indeg * (P2[...] + bpm[...]))
    Wpuv = Wpu[...]
    pn = _dd(pv, Wpuv[0:64]) + _dd(p_agg, Wpuv[64:128]) + bpu[...]
    p_new[...] = pn
    Wmn = Wm_next[...]
    A_next[:, 0:H] = _dd(h_new[...], Wmn[0:64]) + _dd(pn, Wmn[64:128])
    A_next[:, H:D] = jnp.zeros((BLK, H), jnp.float32)


_NH = jax.ShapeDtypeStruct((N_NODES, H), jnp.float32)
_TABLE = jax.ShapeDtypeStruct((N_NODES, D), jnp.float32)
_ESTREAM = jax.ShapeDtypeStruct((E_PAD, D), jnp.float32)


def _blk(w):
    return pl.BlockSpec((BLK, w), lambda i: (i, 0))


def _eblk(w):
    return pl.BlockSpec((EBLK, w), lambda i: (i, 0))


def _full(*shape):
    return pl.BlockSpec(shape, lambda i: (0,) * len(shape))


_TCE = pl.pallas_call(
    _tce_body,
    grid=(NEBLK,),
    in_specs=[_eblk(16), _full(16, H), _full(1, H)],
    out_specs=_eblk(D),
    out_shape=_ESTREAM,
)

_TCQ = pl.pallas_call(
    _tcq_body,
    grid=(NEBLK,),
    in_specs=[_eblk(D), _full(H, H), _full(1, H)],
    out_specs=_eblk(H),
    out_shape=jax.ShapeDtypeStruct((E_PAD, H), jnp.float32),
)

_TC0 = pl.pallas_call(
    _tc0_body,
    grid=(NBLK,),
    in_specs=[_blk(128), _blk(16), _full(128, H), _full(1, H),
              _full(16, H), _full(1, H), _full(320, H),
              _full(NACC, D), _full(NACC, D)],
    out_specs=[_blk(H), _blk(H), _blk(D), _blk(D)],
    out_shape=[_NH, _NH, _TABLE, jax.ShapeDtypeStruct((N_NODES, D),
                                                      jnp.float32)],
)

_TCB = pl.pallas_call(
    _tcb_body,
    grid=(NBLK,),
    in_specs=[_blk(H), _blk(H), _blk(H), _blk(D),
              _full(NACC, D), _full(NACC, D),
              _full(320, H), _full(1, H), _full(128, H), _full(1, H),
              _full(192, H), _full(192, H)],
    out_specs=[_blk(H), _blk(D), _blk(D), _blk(H)],
    out_shape=[_NH, _TABLE, _TABLE, _NH],
)

_TCC = pl.pallas_call(
    _tcc_body,
    grid=(NBLK,),
    in_specs=[_blk(H), _blk(H), _blk(D),
              _full(NACC, D), _full(NACC, D), _blk(H),
              _full(192, H), _full(1, H), _full(128, H), _full(1, H),
              _full(320, H)],
    out_specs=[_blk(H), _blk(D), _blk(H)],
    out_shape=[_NH, _TABLE, _NH],
)


def kernel(h, e, p, edge_index, params):
    send = edge_index[0].astype(jnp.int32)
    rec = edge_index[1].astype(jnp.int32)
    n_pad = E_PAD - N_EDGES
    send_pad = jnp.concatenate([send, jnp.zeros((n_pad,), jnp.int32)])
    rec_pad = jnp.concatenate([rec, jnp.full((n_pad,), DUMP_ROW, jnp.int32)])
    send2d = send_pad.reshape(E_PAD // CHUNK, CHUNK)
    rec2d = rec_pad.reshape(E_PAD // CHUNK, CHUNK)
    sendS = send_pad.reshape(NW, NSCH, SCH)
    recS = rec_pad.reshape(NW, NSCH, SCH)
    iotaS = jnp.arange(E_PAD, dtype=jnp.int32).reshape(NW, NSCH, SCH)

    b2 = lambda x: x.reshape(1, H)

    e_pad = jnp.concatenate([e, jnp.zeros((n_pad, 16), jnp.float32)])
    estream = _TCE(e_pad, params["We"], b2(params["be"]))
    Sp = _SCATTER(iotaS, recS, estream)

    h1, p1, A, SD = _TC0(h, p, params["Wh"], b2(params["bh"]),
                         params["Wp"], b2(params["bp"]),
                         params["layers"][0]["Wm"], Sp[0], Sp[1])
    St = SD[:, H:D]
    hcur, pcur = h1, p1

    n_layers = len(params["layers"])
    for li, lp in enumerate(params["layers"]):
        acc_h = _SCATTER(sendS, recS, A)
        hcur, TS, TR, P2 = _TCB(
            hcur, pcur, St, SD, acc_h[0], acc_h[1],
            lp["Wm"], b2(lp["bm"]), lp["Wu"], b2(lp["bu"]),
            lp["Weu"], lp["Wpm"])
        Q = _TCQ(estream, lp["Weu"][128:192], b2(lp["beu"]))
        estream = _EDGE_UPDATE(send2d, rec2d, TS, TR, Q)
        acc2 = _SCATTER(iotaS, recS, estream)
        Wm_next = params["layers"][(li + 1) % n_layers]["Wm"]
        pcur, A, St = _TCC(pcur, hcur, SD, acc2[0], acc2[1], P2,
                           lp["Wpm"], b2(lp["bpm"]), lp["Wpu"], b2(lp["bpu"]),
                           Wm_next)

    return (hcur, pcur)
